# baseline (speedup 1.0000x reference)
import jax
import jax.numpy as jnp
from jax.experimental import pallas as pl
from jax.experimental.pallas import tpu as pltpu

NBINS = 100


def _norm_kernel(t_ref, mean_ref, std_ref, x_ref, o_ref):
    i = pl.program_id(0)
    tb = (t_ref[i] * NBINS).astype(jnp.int32)
    tb = jnp.where(tb == NBINS, NBINS - 1, tb)
    m = mean_ref[tb]
    s = std_ref[tb]
    r = 1.0 / s
    o_ref[...] = (x_ref[...] - m) * r


def kernel(x_t, t, data_mean, data_std):
    B = x_t.shape[0]
    x = x_t.reshape(B, 512, 512)
    grid_spec = pltpu.PrefetchScalarGridSpec(
        num_scalar_prefetch=3,
        grid=(B,),
        in_specs=[pl.BlockSpec((1, 512, 512), lambda i, *_: (i, 0, 0))],
        out_specs=pl.BlockSpec((1, 512, 512), lambda i, *_: (i, 0, 0)),
    )
    out = pl.pallas_call(
        _norm_kernel,
        grid_spec=grid_spec,
        out_shape=jax.ShapeDtypeStruct(x.shape, x.dtype),
        compiler_params=pltpu.CompilerParams(
            dimension_semantics=("arbitrary",),
        ),
    )(t, data_mean, data_std, x)
    return out.reshape(x_t.shape)


# speedup vs baseline: 1.1397x; 1.1397x over previous
import jax
import jax.numpy as jnp
from jax.experimental import pallas as pl
from jax.experimental.pallas import tpu as pltpu

NBINS = 100


ROWS_PER_BLOCK = 8


def _norm_kernel(t_ref, mean_ref, std_ref, x_ref, o_ref):
    i = pl.program_id(0)
    for r in range(ROWS_PER_BLOCK):
        row = i * ROWS_PER_BLOCK + r
        tb = (t_ref[row] * NBINS).astype(jnp.int32)
        tb = jnp.where(tb == NBINS, NBINS - 1, tb)
        m = mean_ref[tb]
        s = std_ref[tb]
        o_ref[r] = (x_ref[r] - m) * (1.0 / s)


def kernel(x_t, t, data_mean, data_std):
    B = x_t.shape[0]
    x = x_t.reshape(B, 512, 512)
    nb = B // ROWS_PER_BLOCK
    grid_spec = pltpu.PrefetchScalarGridSpec(
        num_scalar_prefetch=3,
        grid=(nb,),
        in_specs=[pl.BlockSpec((ROWS_PER_BLOCK, 512, 512), lambda i, *_: (i, 0, 0))],
        out_specs=pl.BlockSpec((ROWS_PER_BLOCK, 512, 512), lambda i, *_: (i, 0, 0)),
    )
    out = pl.pallas_call(
        _norm_kernel,
        grid_spec=grid_spec,
        out_shape=jax.ShapeDtypeStruct(x.shape, x.dtype),
        compiler_params=pltpu.CompilerParams(
            dimension_semantics=("arbitrary",),
        ),
    )(t, data_mean, data_std, x)
    return out.reshape(x_t.shape)
